# baseline (speedup 1.0000x reference)
import jax
import jax.numpy as jnp
import numpy as np
from jax.experimental import pallas as pl
from jax.experimental.pallas import tpu as pltpu

N_POINTS = 8192
D_COORD = 128
K = 32
N_OUT = 64
R_BLOCK = 256
G = 256
C = N_POINTS // G
T = 4
W2 = 512


def _knn_kernel(sig_ref, x_blk_ref, x_all_ref, vals_ref, idx_ref,
                dist_s, cv_s, ci_s, cv2_s, ci2_s, kd_s, ki_s):
    xb = x_blk_ref[...]
    xa = x_all_ref[...]
    sqb = jnp.sum(xb * xb, axis=1)
    sqa = jnp.sum(xa * xa, axis=1)
    prod = jax.lax.dot_general(
        xb, xa, (((1,), (1,)), ((), ())),
        preferred_element_type=jnp.float32)
    d2 = sqb[:, None] + sqa[None, :] - 2.0 * prod
    dist_s[...] = jnp.sqrt(jnp.maximum(d2, 1e-12))

    INF = jnp.float32(jnp.inf)

    def chain_insert(acc_v, acc_i, t_v, t_i):
        for lvl in range(len(acc_v)):
            swap = t_v < acc_v[lvl]
            nv = jnp.where(swap, t_v, acc_v[lvl])
            t_v = jnp.where(swap, acc_v[lvl], t_v)
            ni = jnp.where(swap, t_i, acc_i[lvl])
            t_i = jnp.where(swap, acc_i[lvl], t_i)
            acc_v[lvl] = nv
            acc_i[lvl] = ni

    g_iota = jax.lax.broadcasted_iota(jnp.int32, (R_BLOCK, G), 1)
    acc_v = [jnp.full((R_BLOCK, G), INF, jnp.float32) for _ in range(T)]
    acc_i = [jnp.full((R_BLOCK, G), N_POINTS, jnp.int32) for _ in range(T)]
    for c in range(C):
        chain_insert(acc_v, acc_i,
                     dist_s[:, c * G:(c + 1) * G], g_iota + (c * G))
    for lvl in range(T):
        cv_s[:, lvl * G:(lvl + 1) * G] = acc_v[lvl]
        ci_s[:, lvl * G:(lvl + 1) * G] = acc_i[lvl]
    v_last1 = acc_v[T - 1]

    a2_v = [jnp.full((R_BLOCK, 128), INF, jnp.float32) for _ in range(4)]
    a2_i = [jnp.full((R_BLOCK, 128), N_POINTS, jnp.int32) for _ in range(4)]
    for lvl in range(T):
        for half in range(G // 128):
            sl = slice(half * 128, (half + 1) * 128)
            chain_insert(a2_v, a2_i, acc_v[lvl][:, sl], acc_i[lvl][:, sl])
    v_last2 = a2_v[3]
    for lvl in range(4):
        cv2_s[:, lvl * 128:(lvl + 1) * 128] = a2_v[lvl]
        ci2_s[:, lvl * 128:(lvl + 1) * 128] = a2_i[lvl]

    def s2_body(t, carry):
        f_v, f_i, ptr = carry
        m = jnp.min(f_v, axis=1)
        eq = f_v == m[:, None]
        sel = jnp.min(jnp.where(eq, f_i, N_POINTS), axis=1)
        kd_s[pl.ds(t, 1), :] = m[None, :]
        ki_s[pl.ds(t, 1), :] = sel[None, :]
        win = f_i == sel[:, None]
        p1 = jnp.where(win, ptr + 1, ptr)
        nv, ni = f_v, f_i
        for l in range(1, 4):
            adv = win & (p1 == l)
            nv = jnp.where(adv, cv2_s[:, l * 128:(l + 1) * 128], nv)
            ni = jnp.where(adv, ci2_s[:, l * 128:(l + 1) * 128], ni)
        exh = win & (p1 >= 4)
        nv = jnp.where(exh, INF, nv)
        ni = jnp.where(exh, N_POINTS, ni)
        return nv, ni, p1

    jax.lax.fori_loop(0, K, s2_body,
                      (a2_v[0], a2_i[0],
                       jnp.zeros((R_BLOCK, 128), jnp.int32)),
                      unroll=False)

    tau = kd_s[K - 1, :]

    @pl.when(jnp.any(v_last2 <= tau[:, None]))
    def _medium_fallback():
        def body(t, _):
            cv = cv_s[...]
            ci = ci_s[...]
            m = jnp.min(cv, axis=1)
            eq = cv == m[:, None]
            sel = jnp.min(jnp.where(eq, ci, N_POINTS), axis=1)
            kd_s[pl.ds(t, 1), :] = m[None, :]
            ki_s[pl.ds(t, 1), :] = sel[None, :]
            cv_s[...] = jnp.where(eq & (ci == sel[:, None]), INF, cv)
            return 0
        jax.lax.fori_loop(0, K, body, 0, unroll=False)

    tau1 = kd_s[K - 1, :]

    @pl.when(jnp.any(v_last1 <= tau1[:, None]))
    def _full_fallback():
        def fb_body(t, _):
            d = dist_s[...]
            lane = jax.lax.broadcasted_iota(jnp.int32, d.shape, 1)
            m = jnp.min(d, axis=1)
            is_m = d == m[:, None]
            idx = jnp.min(jnp.where(is_m, lane, N_POINTS), axis=1)
            kd_s[pl.ds(t, 1), :] = m[None, :]
            ki_s[pl.ds(t, 1), :] = idx[None, :]
            dist_s[...] = jnp.where(lane == idx[:, None], INF, d)
            return 0
        jax.lax.fori_loop(0, K, fb_body, 0, unroll=False)

    idx_ref[...] = ki_s[...]
    kd = kd_s[...].T
    sig = sig_ref[0, :]
    dr = jnp.exp(-(kd[:, :, None] ** 2) / (sig[None, None, :] ** 2 * 2.0))
    vals = jnp.where(dr > 0.1, dr, 0.0)
    vals_ref[...] = vals.reshape(R_BLOCK * K, N_OUT)


def _build_pallas():
    n_blocks = N_POINTS // R_BLOCK
    return pl.pallas_call(
        _knn_kernel,
        grid=(n_blocks,),
        in_specs=[
            pl.BlockSpec((1, N_OUT), lambda i: (0, 0)),
            pl.BlockSpec((R_BLOCK, D_COORD), lambda i: (i, 0)),
            pl.BlockSpec((N_POINTS, D_COORD), lambda i: (0, 0)),
        ],
        out_specs=[
            pl.BlockSpec((R_BLOCK * K, N_OUT), lambda i: (i, 0)),
            pl.BlockSpec((K, R_BLOCK), lambda i: (0, i)),
        ],
        out_shape=[
            jax.ShapeDtypeStruct((N_POINTS * K, N_OUT), jnp.float32),
            jax.ShapeDtypeStruct((K, N_POINTS), jnp.int32),
        ],
        scratch_shapes=[
            pltpu.VMEM((R_BLOCK, N_POINTS), jnp.float32),
            pltpu.VMEM((R_BLOCK, T * G), jnp.float32),
            pltpu.VMEM((R_BLOCK, T * G), jnp.int32),
            pltpu.VMEM((R_BLOCK, W2), jnp.float32),
            pltpu.VMEM((R_BLOCK, W2), jnp.int32),
            pltpu.VMEM((K, R_BLOCK), jnp.float32),
            pltpu.VMEM((K, R_BLOCK), jnp.int32),
        ],
        compiler_params=pltpu.CompilerParams(
            dimension_semantics=("parallel",),
        ),
    )


def kernel(input_coord):
    sig_range = jnp.linspace(0.5, 5.0, N_OUT, dtype=jnp.float32)[None, :]
    vals, ki = _build_pallas()(sig_range, input_coord, input_coord)
    col = ki.T.reshape(-1).astype(jnp.int64)
    row = jnp.repeat(jnp.arange(N_POINTS, dtype=jnp.int64), K)
    batch = jnp.zeros_like(col)
    indices = jnp.stack([batch, row, col], axis=0)
    return indices, vals


# speedup vs baseline: 1.1798x; 1.1798x over previous
import jax
import jax.numpy as jnp
import numpy as np
from jax.experimental import pallas as pl
from jax.experimental.pallas import tpu as pltpu

N_POINTS = 8192
D_COORD = 128
K = 32
N_OUT = 64
R_BLOCK = 128
G = 256
C = N_POINTS // G
T = 4


def _knn_kernel(sig_ref, x_blk_ref, x_all_ref, vals_ref, idx_ref,
                dist_s, cv_s, ci_s, kd_s, ki_s):
    xb = x_blk_ref[...]
    xa = x_all_ref[...]
    sqb = jnp.sum(xb * xb, axis=1)
    sqa = jnp.sum(xa * xa, axis=1)
    prod = jax.lax.dot_general(
        xb, xa, (((1,), (1,)), ((), ())),
        preferred_element_type=jnp.float32)
    d2 = sqb[:, None] + sqa[None, :] - 2.0 * prod
    dist_s[...] = jnp.sqrt(jnp.maximum(d2, 1e-12))

    INF = jnp.float32(jnp.inf)
    io32 = jax.lax.broadcasted_iota(jnp.int32, (R_BLOCK, K), 1)

    def chain_insert(acc_v, acc_i, t_v, t_i):
        for lvl in range(len(acc_v)):
            swap = t_v < acc_v[lvl]
            nv = jnp.where(swap, t_v, acc_v[lvl])
            t_v = jnp.where(swap, acc_v[lvl], t_v)
            ni = jnp.where(swap, t_i, acc_i[lvl])
            t_i = jnp.where(swap, acc_i[lvl], t_i)
            acc_v[lvl] = nv
            acc_i[lvl] = ni

    g_iota = jax.lax.broadcasted_iota(jnp.int32, (R_BLOCK, G), 1)
    acc_v = [jnp.full((R_BLOCK, G), INF, jnp.float32) for _ in range(T)]
    acc_i = [jnp.full((R_BLOCK, G), N_POINTS, jnp.int32) for _ in range(T)]
    for c in range(C):
        chain_insert(acc_v, acc_i,
                     dist_s[:, c * G:(c + 1) * G], g_iota + (c * G))
    for lvl in range(T):
        cv_s[:, lvl * G:(lvl + 1) * G] = acc_v[lvl]
        ci_s[:, lvl * G:(lvl + 1) * G] = acc_i[lvl]
    v_last1 = acc_v[T - 1]

    lv = [jnp.full((R_BLOCK, 128), INF, jnp.float32) for _ in range(4)]
    li = [jnp.full((R_BLOCK, 128), N_POINTS, jnp.int32) for _ in range(4)]
    for lvl in range(T):
        for half in range(G // 128):
            sl = slice(half * 128, (half + 1) * 128)
            chain_insert(lv, li, acc_v[lvl][:, sl], acc_i[lvl][:, sl])
    v_last2 = lv[3]

    kd_acc = jnp.zeros((R_BLOCK, K), jnp.float32)
    ki_acc = jnp.zeros((R_BLOCK, K), jnp.int32)
    for t in range(K):
        m = jnp.min(lv[0], axis=1, keepdims=True)
        eq = lv[0] == m
        sel = jnp.min(jnp.where(eq, li[0], N_POINTS), axis=1,
                      keepdims=True)
        hit = io32 == t
        kd_acc = jnp.where(hit, m, kd_acc)
        ki_acc = jnp.where(hit, sel, ki_acc)
        win = li[0] == sel
        for l in range(3):
            lv[l] = jnp.where(win, lv[l + 1], lv[l])
            li[l] = jnp.where(win, li[l + 1], li[l])
        lv[3] = jnp.where(win, INF, lv[3])
        li[3] = jnp.where(win, N_POINTS, li[3])
    kd_s[...] = kd_acc
    ki_s[...] = ki_acc

    tau = jnp.max(kd_acc, axis=1, keepdims=True)

    @pl.when(jnp.any(v_last2 <= tau))
    def _medium_fallback():
        def body(t, c):
            kd, ki = c
            cv = cv_s[...]
            ci = ci_s[...]
            m = jnp.min(cv, axis=1, keepdims=True)
            eq = cv == m
            sel = jnp.min(jnp.where(eq, ci, N_POINTS), axis=1,
                          keepdims=True)
            kd = jnp.where(io32 == t, m, kd)
            ki = jnp.where(io32 == t, sel, ki)
            cv_s[...] = jnp.where(eq & (ci == sel), INF, cv)
            return kd, ki
        kd2, ki2 = jax.lax.fori_loop(0, K, body, (kd_acc, ki_acc),
                                     unroll=False)
        kd_s[...] = kd2
        ki_s[...] = ki2

    tau1 = jnp.max(kd_s[...], axis=1, keepdims=True)

    @pl.when(jnp.any(v_last1 <= tau1))
    def _full_fallback():
        def fb_body(t, c):
            kd, ki = c
            d = dist_s[...]
            lane = jax.lax.broadcasted_iota(jnp.int32, d.shape, 1)
            m = jnp.min(d, axis=1, keepdims=True)
            is_m = d == m
            idx = jnp.min(jnp.where(is_m, lane, N_POINTS), axis=1,
                          keepdims=True)
            kd = jnp.where(io32 == t, m, kd)
            ki = jnp.where(io32 == t, idx, ki)
            dist_s[...] = jnp.where(lane == idx, INF, d)
            return kd, ki
        kd3, ki3 = jax.lax.fori_loop(0, K, fb_body, (kd_acc, ki_acc),
                                     unroll=False)
        kd_s[...] = kd3
        ki_s[...] = ki3

    kd = kd_s[...]
    idx_ref[...] = ki_s[...]
    sig = sig_ref[0, :]
    dr = jnp.exp(-(kd[:, :, None] ** 2) / (sig[None, None, :] ** 2 * 2.0))
    vals = jnp.where(dr > 0.1, dr, 0.0)
    vals_ref[...] = vals.reshape(R_BLOCK * K, N_OUT)


def _build_pallas():
    n_blocks = N_POINTS // R_BLOCK
    return pl.pallas_call(
        _knn_kernel,
        grid=(n_blocks,),
        in_specs=[
            pl.BlockSpec((1, N_OUT), lambda i: (0, 0)),
            pl.BlockSpec((R_BLOCK, D_COORD), lambda i: (i, 0)),
            pl.BlockSpec((N_POINTS, D_COORD), lambda i: (0, 0)),
        ],
        out_specs=[
            pl.BlockSpec((R_BLOCK * K, N_OUT), lambda i: (i, 0)),
            pl.BlockSpec((R_BLOCK, K), lambda i: (i, 0)),
        ],
        out_shape=[
            jax.ShapeDtypeStruct((N_POINTS * K, N_OUT), jnp.float32),
            jax.ShapeDtypeStruct((N_POINTS, K), jnp.int32),
        ],
        scratch_shapes=[
            pltpu.VMEM((R_BLOCK, N_POINTS), jnp.float32),
            pltpu.VMEM((R_BLOCK, T * G), jnp.float32),
            pltpu.VMEM((R_BLOCK, T * G), jnp.int32),
            pltpu.VMEM((R_BLOCK, K), jnp.float32),
            pltpu.VMEM((R_BLOCK, K), jnp.int32),
        ],
        compiler_params=pltpu.CompilerParams(
            dimension_semantics=("parallel",),
        ),
    )


def kernel(input_coord):
    sig_range = jnp.linspace(0.5, 5.0, N_OUT, dtype=jnp.float32)[None, :]
    vals, ki = _build_pallas()(sig_range, input_coord, input_coord)
    col = ki.reshape(-1).astype(jnp.int64)
    row = jnp.repeat(jnp.arange(N_POINTS, dtype=jnp.int64), K)
    batch = jnp.zeros_like(col)
    indices = jnp.stack([batch, row, col], axis=0)
    return indices, vals
